# baseline (speedup 1.0000x reference)
import functools

import jax
import jax.numpy as jnp
from jax import lax
from jax.experimental import pallas as pl
from jax.experimental.pallas import tpu as pltpu
from jax.experimental.pallas import tpu_sc as plsc

_L = 16
_U = 2


def _sc_fused(idx, ig, emb_flat, pp_flat, w1b, w2b, w3b):
    B = idx.shape[0]
    V8 = emb_flat.shape[0]
    V3 = pp_flat.shape[0]
    info = plsc.get_sparse_core_info()
    nc = info.num_cores
    nw = nc * info.num_subcores
    bw = B // nw
    iters = bw // (_U * _L)
    mesh = plsc.VectorSubcoreMesh(core_axis_name="c", subcore_axis_name="s")

    @functools.partial(
        pl.kernel,
        mesh=mesh,
        compiler_params=pltpu.CompilerParams(
            use_tc_tiling_on_sc=False, needs_layout_passes=False),
        out_type=jax.ShapeDtypeStruct((B, 2), jnp.float32),
        scratch_types=[
            pltpu.VMEM((bw,), jnp.int32),
            pltpu.VMEM((bw,), jnp.float32),
            pltpu.VMEM((V8,), jnp.float32),
            pltpu.VMEM((V3,), jnp.float32),
            pltpu.VMEM((w1b.shape[0],), jnp.float32),
            pltpu.VMEM((w2b.shape[0],), jnp.float32),
            pltpu.VMEM((w3b.shape[0],), jnp.float32),
            pltpu.VMEM((bw, 2), jnp.float32),
        ],
    )
    def fused(idx_hbm, ig_hbm, emb_hbm, pp_hbm, w1_hbm, w2_hbm, w3_hbm,
              out_hbm, idx_v, ig_v, emb_v, pp_v, w1_v, w2_v, w3_v, out_v):
        wid = lax.axis_index("s") * nc + lax.axis_index("c")
        base = wid * bw
        pltpu.sync_copy(idx_hbm.at[pl.ds(base, bw)], idx_v)
        pltpu.sync_copy(ig_hbm.at[pl.ds(base, bw)], ig_v)
        pltpu.sync_copy(emb_hbm, emb_v)
        pltpu.sync_copy(pp_hbm, pp_v)
        pltpu.sync_copy(w1_hbm, w1_v)
        pltpu.sync_copy(w2_hbm, w2_v)
        pltpu.sync_copy(w3_hbm, w3_v)

        iota = lax.broadcasted_iota(jnp.int32, (_L,), 0)

        w1 = lambda s: w1_v[pl.ds(s * _L, _L)]
        w2 = lambda s: w2_v[pl.ds(s * _L, _L)]
        w3 = lambda s: w3_v[pl.ds(s * _L, _L)]

        @plsc.parallel_loop(0, iters)
        def body(g):
            off0 = g * (_U * _L)
            igs, idxs, embs = [], [], []
            for u in range(_U):
                iv = idx_v[pl.ds(off0 + u * _L, _L)]
                idxs.append(iv)
                igs.append(ig_v[pl.ds(off0 + u * _L, _L)])
                i8 = iv * 8
                embs.append([plsc.load_gather(emb_v, [i8 + d])
                             for d in range(8)])

            acc = {}
            for j in range(32):
                ws = [w1(j * 10 + d) for d in range(10)]
                for u in range(_U):
                    a = embs[u][0] * ws[0] + ws[9]
                    for d in range(1, 8):
                        a = a + embs[u][d] * ws[d]
                    a = jnp.maximum(a + igs[u] * ws[8], 0.0)
                    for k in range(16):
                        wkj = w2(k * 33 + j)
                        if j == 0:
                            acc[(k, u)] = a * wkj
                        else:
                            acc[(k, u)] = acc[(k, u)] + a * wkj

            outx, outy = [], []
            for u in range(_U):
                i3 = idxs[u] * 3
                outx.append(plsc.load_gather(pp_v, [i3]) + w3(32))
                outy.append(plsc.load_gather(pp_v, [i3 + 1]) + w3(33))

            for k in range(16):
                bk = w2(k * 33 + 32)
                w3x = w3(k * 2)
                w3y = w3(k * 2 + 1)
                for u in range(_U):
                    r = jnp.maximum(acc[(k, u)] + bk, 0.0)
                    outx[u] = outx[u] + r * w3x
                    outy[u] = outy[u] + r * w3y

            zc = iota * 0
            for u in range(_U):
                rows = iota + (off0 + u * _L)
                plsc.store_scatter(out_v, [rows, zc], outx[u])
                plsc.store_scatter(out_v, [rows, zc + 1], outy[u])

        pltpu.sync_copy(out_v, out_hbm.at[pl.ds(base, bw)])

    return fused(idx, ig, emb_flat, pp_flat, w1b, w2b, w3b)


def _broadcast16(x):
    return jnp.repeat(x.reshape(-1), _L)


def kernel(action_idx, is_ground, physics_params, action_emb,
           W1, b1, W2, b2, W3, b3, gravity):
    idx = action_idx.astype(jnp.int32)
    w1b = _broadcast16(jnp.concatenate([W1, b1[:, None]], axis=1))
    w2b = _broadcast16(jnp.concatenate([W2, b2[:, None]], axis=1))
    w3b = _broadcast16(jnp.concatenate([W3.T.reshape(-1), b3]))
    out = _sc_fused(idx, is_ground, action_emb.reshape(-1),
                    physics_params.reshape(-1), w1b, w2b, w3b)
    return (out, gravity)


# speedup vs baseline: 1.7544x; 1.7544x over previous
import functools

import jax
import jax.numpy as jnp
from jax import lax
from jax.experimental import pallas as pl
from jax.experimental.pallas import tpu as pltpu
from jax.experimental.pallas import tpu_sc as plsc

_TBL_W = 16


def _sc_gather(table, idx):
    V, D = table.shape
    B = idx.shape[0]
    info = plsc.get_sparse_core_info()
    nw = info.num_cores * info.num_subcores
    b_per_w = B // nw
    mesh = plsc.VectorSubcoreMesh(core_axis_name="c", subcore_axis_name="s")

    @functools.partial(
        pl.kernel,
        mesh=mesh,
        compiler_params=pltpu.CompilerParams(use_tc_tiling_on_sc=False),
        out_type=jax.ShapeDtypeStruct((B, D), jnp.float32),
        scratch_types=[
            pltpu.VMEM((b_per_w,), jnp.int32),
            pltpu.VMEM((b_per_w, D), jnp.float32),
            pltpu.SemaphoreType.DMA,
        ],
    )
    def gather_kernel(table_hbm, idx_hbm, out_hbm, idx_v, rows_v, sem):
        wid = lax.axis_index("s") * info.num_cores + lax.axis_index("c")
        base = wid * b_per_w
        pltpu.sync_copy(idx_hbm.at[pl.ds(base, b_per_w)], idx_v)
        pltpu.async_copy(table_hbm.at[idx_v], rows_v, sem).wait()
        pltpu.sync_copy(rows_v, out_hbm.at[pl.ds(base, b_per_w)])

    return gather_kernel(table, idx)


def _tc_mlp(g, ig, W1, b1, W2, b2, W3, b3):
    B = g.shape[0]
    blk = 8192
    grid = (B // blk,)

    def body(g_ref, ig_ref, w1_ref, b1_ref, w2_ref, b2_ref, w3_ref,
             b3_ref, out_ref):
        x = g_ref[...]
        w1 = w1_ref[...]
        emb = x[:, 2:10]
        dn = (((1,), (1,)), ((), ()))
        h = lax.dot_general(emb, w1[:, :8], dn,
                            preferred_element_type=jnp.float32)
        h = h + ig_ref[...] * w1[:, 8][None, :] + b1_ref[...]
        h = jnp.maximum(h, 0.0)
        h = lax.dot_general(h, w2_ref[...], dn,
                            preferred_element_type=jnp.float32)
        h = jnp.maximum(h + b2_ref[...], 0.0)
        res = lax.dot_general(h, w3_ref[...], dn,
                              preferred_element_type=jnp.float32)
        out_ref[...] = x[:, 0:2] + res + b3_ref[...]

    full = lambda shape: pl.BlockSpec(shape, lambda i: (0, 0))
    return pl.pallas_call(
        body,
        grid=grid,
        in_specs=[
            pl.BlockSpec((blk, _TBL_W), lambda i: (i, 0)),
            pl.BlockSpec((blk, 1), lambda i: (i, 0)),
            full((32, 9)),
            full((1, 32)),
            full((16, 32)),
            full((1, 16)),
            full((2, 16)),
            full((1, 2)),
        ],
        out_specs=pl.BlockSpec((blk, 2), lambda i: (i, 0)),
        out_shape=jax.ShapeDtypeStruct((B, 2), jnp.float32),
    )(g, ig, W1, b1, W2, b2, W3, b3)


def kernel(action_idx, is_ground, physics_params, action_emb,
           W1, b1, W2, b2, W3, b3, gravity):
    B = action_idx.shape[0]
    V = physics_params.shape[0]
    idx = action_idx.astype(jnp.int32)
    table = jnp.concatenate(
        [physics_params[:, :2], action_emb,
         jnp.zeros((V, _TBL_W - 10), jnp.float32)], axis=1)
    g = _sc_gather(table, idx)
    out = _tc_mlp(g, is_ground.reshape(B, 1), W1, b1.reshape(1, 32),
                  W2, b2.reshape(1, 16), W3, b3.reshape(1, 2))
    return (out, gravity)
